# baseline (speedup 1.0000x reference)
import functools

import jax
import jax.numpy as jnp
from jax import lax
from jax.experimental import pallas as pl
from jax.experimental.pallas import tpu as pltpu
from jax.experimental.pallas import tpu_sc as plsc

B = 1024
L = 200
D = 128
NEIGH = 256
V = 100000
M = B * L

CHUNKS = (256, 384, 384)
NCHUNK = len(CHUNKS)


_NC = 2
_NS = 16
_NW = _NC * _NS
_G = 80


def _make_sc_gather(bc, nbuf):
    mc = bc * L
    pw = mc // _NW
    ng = pw // _G
    nblk = ng // nbuf
    assert ng == nblk * nbuf and pw == ng * _G

    def body_fn(table_hbm, idx_hbm, out_hbm, idx_v, *rs):
        rows = rs[:nbuf]
        sems = rs[nbuf:]
        wid = lax.axis_index("s") * _NC + lax.axis_index("c")
        base = wid * pw
        pltpu.sync_copy(idx_hbm.at[wid], idx_v)

        for p in range(nbuf):
            pltpu.async_copy(table_hbm.at[idx_v.at[p]], rows[p], sems[p])

        def body(t, carry):
            for p in range(nbuf):
                g = t * nbuf + p
                pltpu.make_async_copy(table_hbm.at[idx_v.at[g]], rows[p],
                                      sems[p]).wait()
                pltpu.sync_copy(rows[p], out_hbm.at[pl.ds(base + g * _G, _G)])
                pltpu.async_copy(table_hbm.at[idx_v.at[g + nbuf]], rows[p],
                                 sems[p])
            return carry

        lax.fori_loop(0, nblk - 1, body, 0, unroll=False)

        for p in range(nbuf):
            g = (nblk - 1) * nbuf + p
            pltpu.make_async_copy(table_hbm.at[idx_v.at[g]], rows[p],
                                  sems[p]).wait()
            pltpu.sync_copy(rows[p], out_hbm.at[pl.ds(base + g * _G, _G)])

    def run(token_table, idx3):
        mesh = plsc.VectorSubcoreMesh(core_axis_name="c", subcore_axis_name="s")
        return pl.kernel(
            body_fn,
            out_type=jax.ShapeDtypeStruct((mc, D), jnp.float32),
            mesh=mesh,
            scratch_types=(
                [pltpu.VMEM((ng, _G), jnp.int32)]
                + [pltpu.VMEM((_G, D), jnp.float32) for _ in range(nbuf)]
                + [pltpu.SemaphoreType.DMA for _ in range(nbuf)]
            ),
        )(token_table, idx3)

    return run


_BB = 32


def _tc_body(nv_ref, nh_ref, tok_ref, pos_ref, wp_ref, bp_ref, we_ref, be_ref,
             wp1_ref, bp1_ref, we1_ref, be1_ref, out_ref, wv_s, wh_s, pb_s):
    @pl.when(pl.program_id(0) == 0)
    def _fold():
        wv_s[...] = jnp.dot(wp_ref[...], we_ref[...],
                            preferred_element_type=jnp.float32)
        wh_s[...] = jnp.dot(wp1_ref[...], we1_ref[...],
                            preferred_element_type=jnp.float32)
        bias = (jnp.dot(bp_ref[...], we_ref[...],
                        preferred_element_type=jnp.float32)
                + be_ref[...]
                + jnp.dot(bp1_ref[...], we1_ref[...],
                          preferred_element_type=jnp.float32)
                + be1_ref[...])
        pb_s[...] = pos_ref[...] + bias

    xv = nv_ref[...].reshape(_BB * L, NEIGH).astype(jnp.bfloat16)
    xh = nh_ref[...].reshape(_BB * L, NEIGH).astype(jnp.bfloat16)
    acc = jnp.dot(xv, wv_s[...].astype(jnp.bfloat16),
                  preferred_element_type=jnp.float32)
    acc = acc + jnp.dot(xh, wh_s[...].astype(jnp.bfloat16),
                        preferred_element_type=jnp.float32)
    out_ref[...] = acc.reshape(_BB, L, D) + tok_ref[...] + pb_s[...][None, :, :]


def _tc_body_alias(nv_ref, nh_ref, tok_ref, pos_ref, wp_ref, bp_ref, we_ref,
                   be_ref, wp1_ref, bp1_ref, we1_ref, be1_ref, prev_ref,
                   out_ref, wv_s, wh_s, pb_s):
    del prev_ref
    _tc_body(nv_ref, nh_ref, tok_ref, pos_ref, wp_ref, bp_ref, we_ref, be_ref,
             wp1_ref, bp1_ref, we1_ref, be1_ref, out_ref, wv_s, wh_s, pb_s)


def _tc_dense_chunk(k, boff, bc, out_prev, nv, nh, tok_c, pos_table, W_prior,
                    b_prior, W_emb, b_emb, W_prior1, b_prior1, W_emb1, b_emb1):
    grid = (bc // _BB,)
    off = boff // _BB
    blk = lambda i: (off + i, 0, 0)
    blk_c = lambda i: (i, 0, 0)
    rep2 = lambda i: (0, 0)
    in_specs = [
        pl.BlockSpec((_BB, L, NEIGH), blk),
        pl.BlockSpec((_BB, L, NEIGH), blk),
        pl.BlockSpec((_BB, L, D), blk_c),
        pl.BlockSpec((L, D), rep2),
        pl.BlockSpec((NEIGH, D), rep2),
        pl.BlockSpec((1, D), rep2),
        pl.BlockSpec((D, D), rep2),
        pl.BlockSpec((1, D), rep2),
        pl.BlockSpec((NEIGH, D), rep2),
        pl.BlockSpec((1, D), rep2),
        pl.BlockSpec((D, D), rep2),
        pl.BlockSpec((1, D), rep2),
    ]
    args = [nv, nh, tok_c, pos_table, W_prior, b_prior, W_emb, b_emb,
            W_prior1, b_prior1, W_emb1, b_emb1]
    if k == 0:
        body = _tc_body
        aliases = {}
    else:
        body = _tc_body_alias
        in_specs = in_specs + [pl.BlockSpec(memory_space=pl.ANY)]
        args = args + [out_prev]
        aliases = {12: 0}
    return pl.pallas_call(
        body,
        grid=grid,
        in_specs=in_specs,
        out_specs=pl.BlockSpec((_BB, L, D), blk),
        out_shape=jax.ShapeDtypeStruct((B, L, D), jnp.float32),
        input_output_aliases=aliases,
        scratch_shapes=[
            pltpu.VMEM((NEIGH, D), jnp.float32),
            pltpu.VMEM((NEIGH, D), jnp.float32),
            pltpu.VMEM((L, D), jnp.float32),
        ],
    )(*args)


_GATHERS = {bc: _make_sc_gather(bc, 10 if bc <= 128 else 5)
            for bc in set(CHUNKS)}


def kernel(inptxtFeats, inpCoords, neigh_vert, neigh_hor, token_table,
           pos_table, W_prior, b_prior, W_emb, b_emb, W_prior1, b_prior1,
           W_emb1, b_emb1):
    del inpCoords
    idx_flat = inptxtFeats.reshape(M)

    toks = []
    boff = 0
    for bc in CHUNKS:
        mc = bc * L
        idx3 = lax.slice(idx_flat, (boff * L,), (boff * L + mc,)).reshape(
            _NW, mc // (_NW * _G), _G)
        toks.append(_GATHERS[bc](token_table, idx3))
        boff += bc

    bp = b_prior.reshape(1, D)
    be = b_emb.reshape(1, D)
    bp1 = b_prior1.reshape(1, D)
    be1 = b_emb1.reshape(1, D)

    out = None
    boff = 0
    for k, bc in enumerate(CHUNKS):
        tok_c = toks[k].reshape(bc, L, D)
        out = _tc_dense_chunk(k, boff, bc, out, neigh_vert, neigh_hor, tok_c,
                              pos_table, W_prior, bp, W_emb, be, W_prior1,
                              bp1, W_emb1, be1)
        boff += bc
    return out


# speedup vs baseline: 1.0047x; 1.0047x over previous
import functools

import jax
import jax.numpy as jnp
from jax import lax
from jax.experimental import pallas as pl
from jax.experimental.pallas import tpu as pltpu
from jax.experimental.pallas import tpu_sc as plsc

B = 1024
L = 200
D = 128
NEIGH = 256
V = 100000
M = B * L

CHUNKS = (512, 512)
NCHUNK = len(CHUNKS)


_NC = 2
_NS = 16
_NW = _NC * _NS
_G = 128


def _make_sc_gather(bc, nbuf):
    mc = bc * L
    pw = mc // _NW
    ng = pw // _G
    nblk = ng // nbuf
    assert ng == nblk * nbuf and pw == ng * _G

    def body_fn(table_hbm, idx_hbm, out_hbm, idx_v, *rs):
        rows = rs[:nbuf]
        sems = rs[nbuf:]
        wid = lax.axis_index("s") * _NC + lax.axis_index("c")
        base = wid * pw
        pltpu.sync_copy(idx_hbm.at[wid], idx_v)

        for p in range(nbuf):
            pltpu.async_copy(table_hbm.at[idx_v.at[p]], rows[p], sems[p])

        def body(t, carry):
            for p in range(nbuf):
                g = t * nbuf + p
                pltpu.make_async_copy(table_hbm.at[idx_v.at[g]], rows[p],
                                      sems[p]).wait()
                pltpu.sync_copy(rows[p], out_hbm.at[pl.ds(base + g * _G, _G)])
                pltpu.async_copy(table_hbm.at[idx_v.at[g + nbuf]], rows[p],
                                 sems[p])
            return carry

        lax.fori_loop(0, nblk - 1, body, 0, unroll=False)

        for p in range(nbuf):
            g = (nblk - 1) * nbuf + p
            pltpu.make_async_copy(table_hbm.at[idx_v.at[g]], rows[p],
                                  sems[p]).wait()
            pltpu.sync_copy(rows[p], out_hbm.at[pl.ds(base + g * _G, _G)])

    def run(token_table, idx3):
        mesh = plsc.VectorSubcoreMesh(core_axis_name="c", subcore_axis_name="s")
        return pl.kernel(
            body_fn,
            out_type=jax.ShapeDtypeStruct((mc, D), jnp.float32),
            mesh=mesh,
            scratch_types=(
                [pltpu.VMEM((ng, _G), jnp.int32)]
                + [pltpu.VMEM((_G, D), jnp.float32) for _ in range(nbuf)]
                + [pltpu.SemaphoreType.DMA for _ in range(nbuf)]
            ),
        )(token_table, idx3)

    return run


_BB = 32


def _tc_body(nv_ref, nh_ref, tok_ref, pos_ref, wp_ref, bp_ref, we_ref, be_ref,
             wp1_ref, bp1_ref, we1_ref, be1_ref, out_ref, wv_s, wh_s, pb_s):
    @pl.when(pl.program_id(0) == 0)
    def _fold():
        wv_s[...] = jnp.dot(wp_ref[...], we_ref[...],
                            preferred_element_type=jnp.float32)
        wh_s[...] = jnp.dot(wp1_ref[...], we1_ref[...],
                            preferred_element_type=jnp.float32)
        bias = (jnp.dot(bp_ref[...], we_ref[...],
                        preferred_element_type=jnp.float32)
                + be_ref[...]
                + jnp.dot(bp1_ref[...], we1_ref[...],
                          preferred_element_type=jnp.float32)
                + be1_ref[...])
        pb_s[...] = pos_ref[...] + bias

    xv = nv_ref[...].reshape(_BB * L, NEIGH).astype(jnp.bfloat16)
    xh = nh_ref[...].reshape(_BB * L, NEIGH).astype(jnp.bfloat16)
    acc = jnp.dot(xv, wv_s[...].astype(jnp.bfloat16),
                  preferred_element_type=jnp.float32)
    acc = acc + jnp.dot(xh, wh_s[...].astype(jnp.bfloat16),
                        preferred_element_type=jnp.float32)
    out_ref[...] = acc.reshape(_BB, L, D) + tok_ref[...] + pb_s[...][None, :, :]


def _tc_body_alias(nv_ref, nh_ref, tok_ref, pos_ref, wp_ref, bp_ref, we_ref,
                   be_ref, wp1_ref, bp1_ref, we1_ref, be1_ref, prev_ref,
                   out_ref, wv_s, wh_s, pb_s):
    del prev_ref
    _tc_body(nv_ref, nh_ref, tok_ref, pos_ref, wp_ref, bp_ref, we_ref, be_ref,
             wp1_ref, bp1_ref, we1_ref, be1_ref, out_ref, wv_s, wh_s, pb_s)


def _tc_dense_chunk(k, boff, bc, out_prev, nv, nh, tok_c, pos_table, W_prior,
                    b_prior, W_emb, b_emb, W_prior1, b_prior1, W_emb1, b_emb1):
    grid = (bc // _BB,)
    off = boff // _BB
    blk = lambda i: (off + i, 0, 0)
    blk_c = lambda i: (i, 0, 0)
    rep2 = lambda i: (0, 0)
    in_specs = [
        pl.BlockSpec((_BB, L, NEIGH), blk),
        pl.BlockSpec((_BB, L, NEIGH), blk),
        pl.BlockSpec((_BB, L, D), blk_c),
        pl.BlockSpec((L, D), rep2),
        pl.BlockSpec((NEIGH, D), rep2),
        pl.BlockSpec((1, D), rep2),
        pl.BlockSpec((D, D), rep2),
        pl.BlockSpec((1, D), rep2),
        pl.BlockSpec((NEIGH, D), rep2),
        pl.BlockSpec((1, D), rep2),
        pl.BlockSpec((D, D), rep2),
        pl.BlockSpec((1, D), rep2),
    ]
    args = [nv, nh, tok_c, pos_table, W_prior, b_prior, W_emb, b_emb,
            W_prior1, b_prior1, W_emb1, b_emb1]
    if k == 0:
        body = _tc_body
        aliases = {}
    else:
        body = _tc_body_alias
        in_specs = in_specs + [pl.BlockSpec(memory_space=pl.ANY)]
        args = args + [out_prev]
        aliases = {12: 0}
    return pl.pallas_call(
        body,
        grid=grid,
        in_specs=in_specs,
        out_specs=pl.BlockSpec((_BB, L, D), blk),
        out_shape=jax.ShapeDtypeStruct((B, L, D), jnp.float32),
        input_output_aliases=aliases,
        scratch_shapes=[
            pltpu.VMEM((NEIGH, D), jnp.float32),
            pltpu.VMEM((NEIGH, D), jnp.float32),
            pltpu.VMEM((L, D), jnp.float32),
        ],
    )(*args)


_GATHERS = {bc: _make_sc_gather(bc, 10 if bc <= 128 else 5)
            for bc in set(CHUNKS)}


def kernel(inptxtFeats, inpCoords, neigh_vert, neigh_hor, token_table,
           pos_table, W_prior, b_prior, W_emb, b_emb, W_prior1, b_prior1,
           W_emb1, b_emb1):
    del inpCoords
    idx_flat = inptxtFeats.reshape(M)

    toks = []
    boff = 0
    for bc in CHUNKS:
        mc = bc * L
        idx3 = lax.slice(idx_flat, (boff * L,), (boff * L + mc,)).reshape(
            _NW, mc // (_NW * _G), _G)
        toks.append(_GATHERS[bc](token_table, idx3))
        boff += bc

    bp = b_prior.reshape(1, D)
    be = b_emb.reshape(1, D)
    bp1 = b_prior1.reshape(1, D)
    be1 = b_emb1.reshape(1, D)

    out = None
    boff = 0
    for k, bc in enumerate(CHUNKS):
        tok_c = toks[k].reshape(bc, L, D)
        out = _tc_dense_chunk(k, boff, bc, out, neigh_vert, neigh_hor, tok_c,
                              pos_table, W_prior, bp, W_emb, be, W_prior1,
                              bp1, W_emb1, be1)
        boff += bc
    return out
